# baseline (speedup 1.0000x reference)
import functools

import jax
import jax.numpy as jnp
from jax import lax
from jax.experimental import pallas as pl
from jax.experimental.pallas import tpu as pltpu
from jax.experimental.pallas import tpu_sc as plsc

N = 50000
NP = 51200
G = 64
E = 800000
EP = 851968
ER = EP // 128
K = 2
KB = 26
NCORE = 2
NSUB = 16
RPC = NP // NSUB
F32 = jnp.float32
BN = 2048
BQ = BN // 4
NBLK = 25
HIGH = lax.Precision.DEFAULT

@functools.lru_cache(maxsize=None)
def _mesh():
    return plsc.VectorSubcoreMesh(core_axis_name="c", subcore_axis_name="s",
                                  num_cores=NCORE, num_subcores=NSUB)


def _edge_pass_body(rps, feat_split, src_hbm, dst_hbm, g_hbm, out_hbm,
                    acc, zb, sbuf, dbuf, rows,
                    gsem0, gsem1, ssem0, ssem1,
                    isem0, isem1, isem2, isem3):
    c = lax.axis_index("c")
    s = lax.axis_index("s")
    nwin = rps // K

    z16 = jnp.zeros((16,), F32)

    def _zrow(i, carry):
        zb[i, pl.ds(0, 16)] = z16
        zb[i, pl.ds(16, 16)] = z16
        return carry
    lax.fori_loop(0, 128, _zrow, 0)

    zbase = s * RPC

    def _zcp(t, carry):
        pltpu.sync_copy(zb, acc.at[pl.ds(zbase + t * 128, 128)])
        return carry
    lax.fori_loop(0, RPC // 128, _zcp, 0)
    plsc.subcore_barrier()

    if feat_split:
        src0 = s * rps
        dst0 = src0
    else:
        src0 = c * (ER // 2) + s * rps
        dst0 = src0

    gsems = (gsem0, gsem1)
    ssems = (ssem0, ssem1)
    isems = (isem0, isem1, isem2, isem3)

    coff = c * NP if feat_split else 0

    def _fire_idx(slot, w):
        pltpu.async_copy(src_hbm.at[pl.ds(src0 + w * K, K)], sbuf.at[slot],
                         isems[slot])
        pltpu.async_copy(dst_hbm.at[pl.ds(dst0 + w * K, K)], dbuf.at[slot],
                         isems[slot])

    def _xform(slot):
        for j in range(K):
            for t in range(8):
                sl = pl.ds(t * 16, 16)
                v = sbuf[slot, j, sl]
                sbuf[slot, j, sl] = ((v & -BN) + ((v & (BQ - 1)) << 2)
                                     + ((v >> 9) & 3) + coff)
                u = dbuf[slot, j, sl]
                dbuf[slot, j, sl] = ((u & -BN) + ((u & (BQ - 1)) << 2)
                                     + ((u >> 9) & 3))

    def _wait_idx(slot, w):
        pltpu.make_async_copy(src_hbm.at[pl.ds(src0 + w * K, K)],
                              sbuf.at[slot], isems[slot]).wait()
        pltpu.make_async_copy(dst_hbm.at[pl.ds(dst0 + w * K, K)],
                              dbuf.at[slot], isems[slot]).wait()
        _xform(slot)

    def _fire_gather(slot, b):
        for j in range(K):
            pltpu.async_copy(g_hbm.at[sbuf.at[slot, j]],
                             rows.at[b, pl.ds(j * 128, 128)], gsems[b])

    def _wait_gather(slot, b):
        for j in range(K):
            pltpu.make_async_copy(g_hbm.at[sbuf.at[slot, j]],
                                  rows.at[b, pl.ds(j * 128, 128)],
                                  gsems[b]).wait()

    def _fire_scatter(slot, b):
        for j in range(K):
            pltpu.async_copy(rows.at[b, pl.ds(j * 128, 128)],
                             acc.at[dbuf.at[slot, j]], ssems[b], add=True)

    def _wait_scatter(slot, b):
        for j in range(K):
            pltpu.make_async_copy(rows.at[b, pl.ds(j * 128, 128)],
                                  acc.at[dbuf.at[slot, j]], ssems[b]).wait()

    _fire_idx(0, 0)
    _fire_idx(1, 1)
    _wait_idx(0, 0)
    _fire_gather(0, 0)

    def _step(w, b4):
        b = b4 % 2
        nb = 1 - b

        @pl.when(w >= 1)
        def _():
            _wait_scatter((b4 - 1) % 4, nb)

        @pl.when(w + 2 < nwin)
        def _():
            _fire_idx((b4 + 2) % 4, w + 2)

        @pl.when(w + 1 < nwin)
        def _():
            _wait_idx((b4 + 1) % 4, w + 1)
            _fire_gather((b4 + 1) % 4, nb)
        _wait_gather(b4, b)
        _fire_scatter(b4, b)

    def _win4(i, carry):
        w0 = i * 4
        for b4 in range(4):
            _step(w0 + b4, b4)
        return carry
    lax.fori_loop(0, nwin // 4, _win4, 0)
    _wait_scatter((nwin - 1) % 4, (nwin - 1) % 2)

    plsc.subcore_barrier()
    pltpu.sync_copy(acc.at[pl.ds(s * RPC, RPC)],
                    out_hbm.at[pl.ds(c * NP + s * RPC, RPC)])


@functools.lru_cache(maxsize=None)
def _make_edge_pass(rps, feat_split):
    return pl.kernel(
        functools.partial(_edge_pass_body, rps, feat_split),
        out_type=jax.ShapeDtypeStruct((NCORE * NP, 32), F32),
        mesh=_mesh(),
        compiler_params=pltpu.CompilerParams(use_tc_tiling_on_sc=False),
        scratch_types=[
            pltpu.VMEM_SHARED((NP, 32), F32),
            pltpu.VMEM((128, 32), F32),
            pltpu.VMEM((4, K, 128), jnp.int32),
            pltpu.VMEM((4, K, 128), jnp.int32),
            pltpu.VMEM((2, K * 128, 32), F32),
            pltpu.SemaphoreType.DMA,
            pltpu.SemaphoreType.DMA,
            pltpu.SemaphoreType.DMA,
            pltpu.SemaphoreType.DMA,
            pltpu.SemaphoreType.DMA,
            pltpu.SemaphoreType.DMA,
            pltpu.SemaphoreType.DMA,
            pltpu.SemaphoreType.DMA,
        ],
    )


def _feat_pass(srcall, dst2d, gf):
    return _make_edge_pass(ER // NSUB, True)(srcall, dst2d, gf)


def _edge_pass(src2d, dst2d, g3):
    return _make_edge_pass(ER // (2 * NSUB), False)(src2d, dst2d, g3)


def _deg_body(dst_hbm, out_hbm, accd, zb1, ob, ib, dsem):
    c = lax.axis_index("c")
    s = lax.axis_index("s")
    z16 = jnp.zeros((16,), F32)
    o16 = jnp.ones((16,), F32)
    for j in range(8):
        zb1[pl.ds(j * 16, 16)] = z16
        ob[pl.ds(j * 16, 16)] = o16

    def _z(t, carry):
        pltpu.sync_copy(zb1, accd.at[pl.ds(s * RPC + t * 128, 128)])
        return carry
    lax.fori_loop(0, RPC // 128, _z, 0)
    plsc.subcore_barrier()

    rps = ER // 32
    base = c * (ER // 2) + s * rps
    pltpu.sync_copy(dst_hbm.at[pl.ds(base, rps)], ib)

    nb = rps // KB

    def _fire(b):
        def _f(w, carry):
            pltpu.async_copy(ob, accd.at[ib.at[w]], dsem, add=True)
            return carry
        lax.fori_loop(b * KB, (b + 1) * KB, _f, 0)

    def _drain():
        def _d(w, carry):
            pltpu.make_async_copy(ob, accd.at[ib.at[0]], dsem).wait()
            return carry
        lax.fori_loop(0, KB, _d, 0)

    _fire(0)

    def _b(b, carry):
        _fire(b)
        _drain()
        return carry
    lax.fori_loop(1, nb, _b, 0)
    _drain()

    plsc.subcore_barrier()
    pltpu.sync_copy(accd.at[pl.ds(s * RPC, RPC)],
                    out_hbm.at[pl.ds(c * NP + s * RPC, RPC)])


@functools.lru_cache(maxsize=None)
def _make_deg_pass():
    return pl.kernel(
        _deg_body,
        out_type=jax.ShapeDtypeStruct((NCORE * NP,), F32),
        mesh=_mesh(),
        compiler_params=pltpu.CompilerParams(use_tc_tiling_on_sc=False),
        scratch_types=[
            pltpu.VMEM_SHARED((NP,), F32),
            pltpu.VMEM((128,), F32),
            pltpu.VMEM((128,), F32),
            pltpu.VMEM((ER // 32, 128), jnp.int32),
            pltpu.SemaphoreType.DMA,
        ],
    )


def _deg_pass(dst2d):
    return _make_deg_pass()(dst2d)


def _pack(a):
    return jnp.concatenate(
        [a[0:BQ], a[BQ:2 * BQ], a[2 * BQ:3 * BQ], a[3 * BQ:4 * BQ]], axis=1)


def _k1_body(x_ref, degp_ref, w1_ref, g_ref, dis_ref):
    i = pl.program_id(0)
    deg = degp_ref[0, :] + degp_ref[1, :] + 1.0
    dis = lax.rsqrt(deg)
    p = jnp.dot(x_ref[...], w1_ref[...], preferred_element_type=F32,
                precision=HIGH)
    rowid = lax.broadcasted_iota(jnp.int32, (BN, 1), 0) + i * BN
    gsc = jnp.where(rowid < N, p * dis[:, None], 0.0)
    g_ref[0] = _pack(gsc[:, :32])
    g_ref[1] = _pack(gsc[:, 32:])
    dis_ref[...] = _pack(jnp.broadcast_to(dis[:, None], (BN, 32)))


def _k1(x_p, degp, W1):
    return pl.pallas_call(
        _k1_body,
        grid=(NBLK,),
        in_specs=[pl.BlockSpec((BN, 128), lambda i: (i, 0)),
                  pl.BlockSpec((2, BN), lambda i: (0, i)),
                  pl.BlockSpec((128, 64), lambda i: (0, 0))],
        out_specs=[pl.BlockSpec((2, BN // 4, 128), lambda i: (0, i, 0)),
                   pl.BlockSpec((BN // 4, 128), lambda i: (i, 0))],
        out_shape=[jax.ShapeDtypeStruct((2, NP // 4, 128), F32),
                   jax.ShapeDtypeStruct((NP // 4, 128), F32)],
    )(x_p, degp, W1)


def _k23_body(hout, acc_ref, g_ref, dis_ref, b_ref, w_ref, out_ref):
    disp = dis_ref[...]
    h0 = jnp.maximum(disp * (acc_ref[0] + g_ref[0]) + b_ref[0, :][None, :],
                     0.0)
    h1 = jnp.maximum(disp * (acc_ref[1] + g_ref[1]) + b_ref[1, :][None, :],
                     0.0)

    def mm(h, w):
        return jnp.dot(h, w, preferred_element_type=F32, precision=HIGH)

    if hout == 64:
        out_ref[0] = (mm(h0, w_ref[0, 0]) + mm(h1, w_ref[1, 0])) * disp
        out_ref[1] = (mm(h0, w_ref[0, 1]) + mm(h1, w_ref[1, 1])) * disp
    else:
        out_ref[...] = (mm(h0, w_ref[0]) + mm(h1, w_ref[1])) * disp


def _k23(acc, g, dis, b, Wbd):
    hout = 64 if Wbd.ndim == 4 else 32
    if hout == 64:
        out_spec = pl.BlockSpec((2, BN // 4, 128), lambda i: (0, i, 0))
        out_shape = jax.ShapeDtypeStruct((2, NP // 4, 128), F32)
        w_spec = pl.BlockSpec((2, 2, 128, 128), lambda i: (0, 0, 0, 0))
    else:
        out_spec = pl.BlockSpec((BN // 4, 128), lambda i: (i, 0))
        out_shape = jax.ShapeDtypeStruct((NP // 4, 128), F32)
        w_spec = pl.BlockSpec((2, 128, 128), lambda i: (0, 0, 0))
    return pl.pallas_call(
        functools.partial(_k23_body, hout),
        grid=(NBLK,),
        in_specs=[pl.BlockSpec((2, BN // 4, 128), lambda i: (0, i, 0)),
                  pl.BlockSpec((2, BN // 4, 128), lambda i: (0, i, 0)),
                  pl.BlockSpec((BN // 4, 128), lambda i: (i, 0)),
                  pl.BlockSpec((2, 128), lambda i: (0, 0)),
                  w_spec],
        out_specs=out_spec,
        out_shape=out_shape,
    )(acc, g, dis, b, Wbd)


def _k4_body(acc_ref, g3_ref, dis_ref, b3_ref, batch_ref, wl_ref, bl_ref,
             head_ref, emb_ref, esum, csum):
    i = pl.program_id(0)
    disp = dis_ref[...]
    h3 = disp * (acc_ref[0] + acc_ref[1] + g3_ref[...]) + b3_ref[...]
    valid = (batch_ref[0, :] < G).astype(F32)[:, None]
    h3 = jnp.where(_pack(jnp.broadcast_to(valid, (BN, 32))) > 0.5, h3, 0.0)
    gids = lax.broadcasted_iota(jnp.int32, (G, BN), 0)
    oh = (gids == batch_ref[0, :][None, :]).astype(F32)
    ps = jnp.zeros((G, 32), F32)
    for m in range(4):
        ps = ps + jnp.dot(oh[:, BQ * m:BQ * (m + 1)],
                          h3[:, 32 * m:32 * (m + 1)],
                          preferred_element_type=F32, precision=HIGH)
    pc = jnp.sum(oh, axis=1, keepdims=True)

    @pl.when(i == 0)
    def _():
        esum[...] = ps
        csum[...] = pc

    @pl.when(i > 0)
    def _():
        esum[...] = esum[...] + ps
        csum[...] = csum[...] + pc

    @pl.when(i == NBLK - 1)
    def _():
        emb = esum[...] / jnp.maximum(csum[...], 1.0)
        emb_ref[...] = emb
        head_ref[...] = jnp.dot(emb, wl_ref[...], preferred_element_type=F32,
                                precision=HIGH) + bl_ref[...]


def _k4(acc3, g3, dis, b3, batch_p, Wl, bl):
    return pl.pallas_call(
        _k4_body,
        grid=(NBLK,),
        in_specs=[pl.BlockSpec((2, BN // 4, 128), lambda i: (0, i, 0)),
                  pl.BlockSpec((BN // 4, 128), lambda i: (i, 0)),
                  pl.BlockSpec((BN // 4, 128), lambda i: (i, 0)),
                  pl.BlockSpec((1, 128), lambda i: (0, 0)),
                  pl.BlockSpec((1, BN), lambda i: (0, i)),
                  pl.BlockSpec((32, 1), lambda i: (0, 0)),
                  pl.BlockSpec((1, 1), lambda i: (0, 0))],
        out_specs=[pl.BlockSpec((G, 1), lambda i: (0, 0)),
                   pl.BlockSpec((G, 32), lambda i: (0, 0))],
        out_shape=[jax.ShapeDtypeStruct((G, 1), F32),
                   jax.ShapeDtypeStruct((G, 32), F32)],
        scratch_shapes=[pltpu.VMEM((G, 32), F32),
                        pltpu.VMEM((G, 1), F32)],
    )(acc3, g3, dis, b3, batch_p, Wl, bl)


def _bd(M):
    return jnp.kron(jnp.eye(4, dtype=M.dtype), M)


def kernel(x, edge_index, batch, W1, b1, W2, b2, W3, b3, Wl, bl):
    src = edge_index[0]
    dst = edge_index[1]
    padn = NP - N
    batch_p = jnp.pad(batch, (0, padn), constant_values=G).reshape(1, NP)
    pade = EP - E
    ar = jnp.arange(pade, dtype=jnp.int32)
    pad_src = (ar * 97) % N
    pad_dst = N + (ar % padn)
    src2d = jnp.concatenate([src, pad_src]).reshape(ER, 128)
    dst2d = jnp.concatenate([dst, pad_dst]).reshape(ER, 128)

    bp1 = jnp.stack([jnp.tile(b1[:32], 4), jnp.tile(b1[32:], 4)])
    bp2 = jnp.stack([jnp.tile(b2[:32], 4), jnp.tile(b2[32:], 4)])
    bp3 = jnp.tile(b3, 4).reshape(1, 128)
    w2bd = jnp.stack([
        jnp.stack([_bd(W2[:32, :32]), _bd(W2[:32, 32:])]),
        jnp.stack([_bd(W2[32:, :32]), _bd(W2[32:, 32:])]),
    ])
    w3bd = jnp.stack([_bd(W3[:32, :]), _bd(W3[32:, :])])

    degp = _deg_pass(dst2d).reshape(2, NP)
    g1p, disp = _k1(x, degp, W1)
    acc1 = _feat_pass(src2d, dst2d, g1p.reshape(2 * NP, 32))
    g2p = _k23(acc1.reshape(2, NP // 4, 128), g1p, disp, bp1, w2bd)
    acc2 = _feat_pass(src2d, dst2d, g2p.reshape(2 * NP, 32))
    g3p = _k23(acc2.reshape(2, NP // 4, 128), g2p, disp, bp2, w3bd)
    acc3 = _edge_pass(src2d, dst2d, g3p.reshape(NP, 32))
    head, emb = _k4(acc3.reshape(2, NP // 4, 128), g3p, disp,
                    bp3, batch_p, Wl, bl.reshape(1, 1))
    return head, emb


# speedup vs baseline: 1.0114x; 1.0114x over previous
import functools

import jax
import jax.numpy as jnp
from jax import lax
from jax.experimental import pallas as pl
from jax.experimental.pallas import tpu as pltpu
from jax.experimental.pallas import tpu_sc as plsc

N = 50000
NP = 51200
G = 64
E = 800000
EP = 851968
ER = EP // 128
K = 2
KB = 26
NCORE = 2
NSUB = 16
RPC = NP // NSUB
F32 = jnp.float32
BN = 2048
BQ = BN // 4
NBLK = 25
HIGH = lax.Precision.DEFAULT

@functools.lru_cache(maxsize=None)
def _mesh():
    return plsc.VectorSubcoreMesh(core_axis_name="c", subcore_axis_name="s",
                                  num_cores=NCORE, num_subcores=NSUB)


def _edge_pass_body(rps, feat_split, src_hbm, dst_hbm, g_hbm, out_hbm,
                    acc, zb, sbuf, dbuf, rows,
                    gsem0, gsem1, ssem0, ssem1,
                    isem0, isem1, isem2, isem3):
    c = lax.axis_index("c")
    s = lax.axis_index("s")
    nwin = rps // K

    z16 = jnp.zeros((16,), F32)

    def _zrow(i, carry):
        zb[i, pl.ds(0, 16)] = z16
        zb[i, pl.ds(16, 16)] = z16
        return carry
    lax.fori_loop(0, 128, _zrow, 0)

    zbase = s * RPC

    def _zcp(t, carry):
        pltpu.sync_copy(zb, acc.at[pl.ds(zbase + t * 128, 128)])
        return carry
    lax.fori_loop(0, RPC // 128, _zcp, 0)
    plsc.subcore_barrier()

    if feat_split:
        src0 = s * rps
        dst0 = src0
    else:
        src0 = c * (ER // 2) + s * rps
        dst0 = src0

    gsems = (gsem0, gsem1)
    ssems = (ssem0, ssem1)
    isems = (isem0, isem1, isem2, isem3)

    coff = c * NP if feat_split else 0

    def _fire_idx(slot, w):
        pltpu.async_copy(src_hbm.at[pl.ds(src0 + w * K, K)], sbuf.at[slot],
                         isems[slot])
        pltpu.async_copy(dst_hbm.at[pl.ds(dst0 + w * K, K)], dbuf.at[slot],
                         isems[slot])

    def _xform(slot):
        if not feat_split:
            return
        for j in range(K):
            for t in range(8):
                sl = pl.ds(t * 16, 16)
                sbuf[slot, j, sl] = sbuf[slot, j, sl] + coff

    def _wait_idx(slot, w):
        pltpu.make_async_copy(src_hbm.at[pl.ds(src0 + w * K, K)],
                              sbuf.at[slot], isems[slot]).wait()
        pltpu.make_async_copy(dst_hbm.at[pl.ds(dst0 + w * K, K)],
                              dbuf.at[slot], isems[slot]).wait()
        _xform(slot)

    def _fire_gather(slot, b):
        for j in range(K):
            pltpu.async_copy(g_hbm.at[sbuf.at[slot, j]],
                             rows.at[b, pl.ds(j * 128, 128)], gsems[b])

    def _wait_gather(slot, b):
        for j in range(K):
            pltpu.make_async_copy(g_hbm.at[sbuf.at[slot, j]],
                                  rows.at[b, pl.ds(j * 128, 128)],
                                  gsems[b]).wait()

    def _fire_scatter(slot, b):
        for j in range(K):
            pltpu.async_copy(rows.at[b, pl.ds(j * 128, 128)],
                             acc.at[dbuf.at[slot, j]], ssems[b], add=True)

    def _wait_scatter(slot, b):
        for j in range(K):
            pltpu.make_async_copy(rows.at[b, pl.ds(j * 128, 128)],
                                  acc.at[dbuf.at[slot, j]], ssems[b]).wait()

    _fire_idx(0, 0)
    _fire_idx(1, 1)
    _wait_idx(0, 0)
    _fire_gather(0, 0)

    def _step(w, b4):
        b = b4 % 2
        nb = 1 - b

        @pl.when(w >= 1)
        def _():
            _wait_scatter((b4 - 1) % 4, nb)

        @pl.when(w + 2 < nwin)
        def _():
            _fire_idx((b4 + 2) % 4, w + 2)

        @pl.when(w + 1 < nwin)
        def _():
            _wait_idx((b4 + 1) % 4, w + 1)
            _fire_gather((b4 + 1) % 4, nb)
        _wait_gather(b4, b)
        _fire_scatter(b4, b)

    def _win4(i, carry):
        w0 = i * 4
        for b4 in range(4):
            _step(w0 + b4, b4)
        return carry
    lax.fori_loop(0, nwin // 4, _win4, 0)
    _wait_scatter((nwin - 1) % 4, (nwin - 1) % 2)

    plsc.subcore_barrier()
    pltpu.sync_copy(acc.at[pl.ds(s * RPC, RPC)],
                    out_hbm.at[pl.ds(c * NP + s * RPC, RPC)])


@functools.lru_cache(maxsize=None)
def _make_edge_pass(rps, feat_split):
    return pl.kernel(
        functools.partial(_edge_pass_body, rps, feat_split),
        out_type=jax.ShapeDtypeStruct((NCORE * NP, 32), F32),
        mesh=_mesh(),
        compiler_params=pltpu.CompilerParams(use_tc_tiling_on_sc=False),
        scratch_types=[
            pltpu.VMEM_SHARED((NP, 32), F32),
            pltpu.VMEM((128, 32), F32),
            pltpu.VMEM((4, K, 128), jnp.int32),
            pltpu.VMEM((4, K, 128), jnp.int32),
            pltpu.VMEM((2, K * 128, 32), F32),
            pltpu.SemaphoreType.DMA,
            pltpu.SemaphoreType.DMA,
            pltpu.SemaphoreType.DMA,
            pltpu.SemaphoreType.DMA,
            pltpu.SemaphoreType.DMA,
            pltpu.SemaphoreType.DMA,
            pltpu.SemaphoreType.DMA,
            pltpu.SemaphoreType.DMA,
        ],
    )


def _feat_pass(srcall, dst2d, gf):
    return _make_edge_pass(ER // NSUB, True)(srcall, dst2d, gf)


def _edge_pass(src2d, dst2d, g3):
    return _make_edge_pass(ER // (2 * NSUB), False)(src2d, dst2d, g3)


def _deg_body(dst_hbm, out_hbm, accd, zb1, ob, ib, dsem):
    c = lax.axis_index("c")
    s = lax.axis_index("s")
    z16 = jnp.zeros((16,), F32)
    o16 = jnp.ones((16,), F32)
    for j in range(8):
        zb1[pl.ds(j * 16, 16)] = z16
        ob[pl.ds(j * 16, 16)] = o16

    def _z(t, carry):
        pltpu.sync_copy(zb1, accd.at[pl.ds(s * RPC + t * 128, 128)])
        return carry
    lax.fori_loop(0, RPC // 128, _z, 0)
    plsc.subcore_barrier()

    rps = ER // 32
    base = c * (ER // 2) + s * rps
    pltpu.sync_copy(dst_hbm.at[pl.ds(base, rps)], ib)

    nb = rps // KB

    def _fire(b):
        def _f(w, carry):
            pltpu.async_copy(ob, accd.at[ib.at[w]], dsem, add=True)
            return carry
        lax.fori_loop(b * KB, (b + 1) * KB, _f, 0)

    def _drain():
        def _d(w, carry):
            pltpu.make_async_copy(ob, accd.at[ib.at[0]], dsem).wait()
            return carry
        lax.fori_loop(0, KB, _d, 0)

    _fire(0)

    def _b(b, carry):
        _fire(b)
        _drain()
        return carry
    lax.fori_loop(1, nb, _b, 0)
    _drain()

    plsc.subcore_barrier()
    pltpu.sync_copy(accd.at[pl.ds(s * RPC, RPC)],
                    out_hbm.at[pl.ds(c * NP + s * RPC, RPC)])


@functools.lru_cache(maxsize=None)
def _make_deg_pass():
    return pl.kernel(
        _deg_body,
        out_type=jax.ShapeDtypeStruct((NCORE * NP,), F32),
        mesh=_mesh(),
        compiler_params=pltpu.CompilerParams(use_tc_tiling_on_sc=False),
        scratch_types=[
            pltpu.VMEM_SHARED((NP,), F32),
            pltpu.VMEM((128,), F32),
            pltpu.VMEM((128,), F32),
            pltpu.VMEM((ER // 32, 128), jnp.int32),
            pltpu.SemaphoreType.DMA,
        ],
    )


def _deg_pass(dst2d):
    return _make_deg_pass()(dst2d)


def _pack(a):
    return jnp.concatenate(
        [a[0:BQ], a[BQ:2 * BQ], a[2 * BQ:3 * BQ], a[3 * BQ:4 * BQ]], axis=1)


def _k1_body(x_ref, degp_ref, w1_ref, g_ref, dis_ref):
    i = pl.program_id(0)
    deg = degp_ref[0, :] + degp_ref[1, :] + 1.0
    dis = lax.rsqrt(deg)
    p = jnp.dot(x_ref[...], w1_ref[...], preferred_element_type=F32,
                precision=HIGH)
    rowid = lax.broadcasted_iota(jnp.int32, (BN, 1), 0) + i * BN
    gsc = jnp.where(rowid < N, p * dis[:, None], 0.0)
    g_ref[0] = _pack(gsc[:, :32])
    g_ref[1] = _pack(gsc[:, 32:])
    dis_ref[...] = _pack(jnp.broadcast_to(dis[:, None], (BN, 32)))


def _k1(x_p, degp, W1):
    return pl.pallas_call(
        _k1_body,
        grid=(NBLK,),
        in_specs=[pl.BlockSpec((BN, 128), lambda i: (i, 0)),
                  pl.BlockSpec((2, BN), lambda i: (0, i)),
                  pl.BlockSpec((128, 64), lambda i: (0, 0))],
        out_specs=[pl.BlockSpec((2, BN // 4, 128), lambda i: (0, i, 0)),
                   pl.BlockSpec((BN // 4, 128), lambda i: (i, 0))],
        out_shape=[jax.ShapeDtypeStruct((2, NP // 4, 128), F32),
                   jax.ShapeDtypeStruct((NP // 4, 128), F32)],
    )(x_p, degp, W1)


def _k23_body(hout, acc_ref, g_ref, dis_ref, b_ref, w_ref, out_ref):
    disp = dis_ref[...]
    h0 = jnp.maximum(disp * (acc_ref[0] + g_ref[0]) + b_ref[0, :][None, :],
                     0.0)
    h1 = jnp.maximum(disp * (acc_ref[1] + g_ref[1]) + b_ref[1, :][None, :],
                     0.0)

    def mm(h, w):
        return jnp.dot(h, w, preferred_element_type=F32, precision=HIGH)

    if hout == 64:
        out_ref[0] = (mm(h0, w_ref[0, 0]) + mm(h1, w_ref[1, 0])) * disp
        out_ref[1] = (mm(h0, w_ref[0, 1]) + mm(h1, w_ref[1, 1])) * disp
    else:
        out_ref[...] = (mm(h0, w_ref[0]) + mm(h1, w_ref[1])) * disp


def _k23(acc, g, dis, b, Wbd):
    hout = 64 if Wbd.ndim == 4 else 32
    if hout == 64:
        out_spec = pl.BlockSpec((2, BN // 4, 128), lambda i: (0, i, 0))
        out_shape = jax.ShapeDtypeStruct((2, NP // 4, 128), F32)
        w_spec = pl.BlockSpec((2, 2, 128, 128), lambda i: (0, 0, 0, 0))
    else:
        out_spec = pl.BlockSpec((BN // 4, 128), lambda i: (i, 0))
        out_shape = jax.ShapeDtypeStruct((NP // 4, 128), F32)
        w_spec = pl.BlockSpec((2, 128, 128), lambda i: (0, 0, 0))
    return pl.pallas_call(
        functools.partial(_k23_body, hout),
        grid=(NBLK,),
        in_specs=[pl.BlockSpec((2, BN // 4, 128), lambda i: (0, i, 0)),
                  pl.BlockSpec((2, BN // 4, 128), lambda i: (0, i, 0)),
                  pl.BlockSpec((BN // 4, 128), lambda i: (i, 0)),
                  pl.BlockSpec((2, 128), lambda i: (0, 0)),
                  w_spec],
        out_specs=out_spec,
        out_shape=out_shape,
    )(acc, g, dis, b, Wbd)


def _k4_body(acc_ref, g3_ref, dis_ref, b3_ref, batch_ref, wl_ref, bl_ref,
             head_ref, emb_ref, esum, csum):
    i = pl.program_id(0)
    disp = dis_ref[...]
    h3 = disp * (acc_ref[0] + acc_ref[1] + g3_ref[...]) + b3_ref[...]
    valid = (batch_ref[0, :] < G).astype(F32)[:, None]
    h3 = jnp.where(_pack(jnp.broadcast_to(valid, (BN, 32))) > 0.5, h3, 0.0)
    gids = lax.broadcasted_iota(jnp.int32, (G, BN), 0)
    oh = (gids == batch_ref[0, :][None, :]).astype(F32)
    ps = jnp.zeros((G, 32), F32)
    for m in range(4):
        ps = ps + jnp.dot(oh[:, BQ * m:BQ * (m + 1)],
                          h3[:, 32 * m:32 * (m + 1)],
                          preferred_element_type=F32, precision=HIGH)
    pc = jnp.sum(oh, axis=1, keepdims=True)

    @pl.when(i == 0)
    def _():
        esum[...] = ps
        csum[...] = pc

    @pl.when(i > 0)
    def _():
        esum[...] = esum[...] + ps
        csum[...] = csum[...] + pc

    @pl.when(i == NBLK - 1)
    def _():
        emb = esum[...] / jnp.maximum(csum[...], 1.0)
        emb_ref[...] = emb
        head_ref[...] = jnp.dot(emb, wl_ref[...], preferred_element_type=F32,
                                precision=HIGH) + bl_ref[...]


def _k4(acc3, g3, dis, b3, batch_p, Wl, bl):
    return pl.pallas_call(
        _k4_body,
        grid=(NBLK,),
        in_specs=[pl.BlockSpec((2, BN // 4, 128), lambda i: (0, i, 0)),
                  pl.BlockSpec((BN // 4, 128), lambda i: (i, 0)),
                  pl.BlockSpec((BN // 4, 128), lambda i: (i, 0)),
                  pl.BlockSpec((1, 128), lambda i: (0, 0)),
                  pl.BlockSpec((1, BN), lambda i: (0, i)),
                  pl.BlockSpec((32, 1), lambda i: (0, 0)),
                  pl.BlockSpec((1, 1), lambda i: (0, 0))],
        out_specs=[pl.BlockSpec((G, 1), lambda i: (0, 0)),
                   pl.BlockSpec((G, 32), lambda i: (0, 0))],
        out_shape=[jax.ShapeDtypeStruct((G, 1), F32),
                   jax.ShapeDtypeStruct((G, 32), F32)],
        scratch_shapes=[pltpu.VMEM((G, 32), F32),
                        pltpu.VMEM((G, 1), F32)],
    )(acc3, g3, dis, b3, batch_p, Wl, bl)


def _perm(n):
    return (n & -BN) + ((n & (BQ - 1)) << 2) + ((n >> 9) & 3)


def _bd(M):
    return jnp.kron(jnp.eye(4, dtype=M.dtype), M)


def kernel(x, edge_index, batch, W1, b1, W2, b2, W3, b3, Wl, bl):
    src = edge_index[0]
    dst = edge_index[1]
    padn = NP - N
    batch_p = jnp.pad(batch, (0, padn), constant_values=G).reshape(1, NP)
    pade = EP - E
    ar = jnp.arange(pade, dtype=jnp.int32)
    pad_src = (ar * 97) % N
    pad_dst = N + (ar % padn)
    srcf = jnp.concatenate([src, pad_src])
    dstf = jnp.concatenate([dst, pad_dst])
    dst2d = dstf.reshape(ER, 128)
    src2dp = _perm(srcf).reshape(ER, 128)
    dst2dp = _perm(dstf).reshape(ER, 128)

    bp1 = jnp.stack([jnp.tile(b1[:32], 4), jnp.tile(b1[32:], 4)])
    bp2 = jnp.stack([jnp.tile(b2[:32], 4), jnp.tile(b2[32:], 4)])
    bp3 = jnp.tile(b3, 4).reshape(1, 128)
    w2bd = jnp.stack([
        jnp.stack([_bd(W2[:32, :32]), _bd(W2[:32, 32:])]),
        jnp.stack([_bd(W2[32:, :32]), _bd(W2[32:, 32:])]),
    ])
    w3bd = jnp.stack([_bd(W3[:32, :]), _bd(W3[32:, :])])

    degp = _deg_pass(dst2d).reshape(2, NP)
    g1p, disp = _k1(x, degp, W1)
    acc1 = _feat_pass(src2dp, dst2dp, g1p.reshape(2 * NP, 32))
    g2p = _k23(acc1.reshape(2, NP // 4, 128), g1p, disp, bp1, w2bd)
    acc2 = _feat_pass(src2dp, dst2dp, g2p.reshape(2 * NP, 32))
    g3p = _k23(acc2.reshape(2, NP // 4, 128), g2p, disp, bp2, w3bd)
    acc3 = _edge_pass(src2dp, dst2dp, g3p.reshape(NP, 32))
    head, emb = _k4(acc3.reshape(2, NP // 4, 128), g3p, disp,
                    bp3, batch_p, Wl, bl.reshape(1, 1))
    return head, emb
